# baseline (speedup 1.0000x reference)
import functools

import jax
import jax.numpy as jnp
from jax import lax
from jax.experimental import pallas as pl
from jax.experimental.pallas import tpu as pltpu
from jax.experimental.pallas import tpu_sc as plsc

_QB = 1024
_BIG = 2 ** 30
_BIGF = 3.0e38
_KSPLIT = 2304


def _argmin_body(n_keys, q_ref, qs_ref, ks_ref, kt_ref, idx_ref, val_ref):
    q = q_ref[0]
    a2 = jnp.sum(q * q, axis=1, keepdims=True)
    k = kt_ref[0]
    b2 = jnp.sum(k * k, axis=0, keepdims=True)
    mm = jnp.dot(qs_ref[0], ks_ref[0], preferred_element_type=jnp.float32)
    m = jnp.min((a2 + b2) + mm, axis=1, keepdims=True)
    mc = jnp.maximum(m, 0.0)
    iota = lax.broadcasted_iota(jnp.int32, (1, mm.shape[1]), 1).astype(jnp.float32)
    idx = jnp.min(jnp.where(((a2 + b2) + mm) <= mc, iota, jnp.float32(_BIG)),
                  axis=1, keepdims=True)
    idx_ref[0] = idx.astype(jnp.int32) + pl.program_id(0) * n_keys
    val_ref[0] = mc


def _nearest_tc(pred, qs, ks, kt):
    B, N, _ = pred.shape
    K = ks.shape[2]
    nb = N // _QB
    idx, val = pl.pallas_call(
        functools.partial(_argmin_body, N),
        grid=(B, nb),
        in_specs=[
            pl.BlockSpec((1, _QB, 2), lambda b, i: (b, i, 0)),
            pl.BlockSpec((1, _QB, 2), lambda b, i: (b, i, 0)),
            pl.BlockSpec((1, 2, K), lambda b, i: (b, 0, 0)),
            pl.BlockSpec((1, 2, K), lambda b, i: (b, 0, 0)),
        ],
        out_specs=[
            pl.BlockSpec((1, _QB, 1), lambda b, i: (b * nb + i, 0, 0)),
            pl.BlockSpec((1, _QB, 1), lambda b, i: (b * nb + i, 0, 0)),
        ],
        out_shape=[
            jax.ShapeDtypeStruct((B * nb, _QB, 1), jnp.int32),
            jax.ShapeDtypeStruct((B * nb, _QB, 1), jnp.float32),
        ],
    )(pred, qs, ks, kt)
    return idx.reshape(B * N), val.reshape(B * N)


_NC = 2
_NS = 16
_NW = _NC * _NS
_QW = 512
_UNROLL = 8


def _round_bf16_vec(x):
    u = lax.bitcast_convert_type(x, jnp.uint32)
    u = (u + jnp.uint32(0x7FFF) + ((u >> 16) & jnp.uint32(1))) & jnp.uint32(0xFFFF0000)
    return lax.bitcast_convert_type(u, jnp.float32)


def _nearest_sc(px, py, kx, ky, n_total, ksc):
    BN = px.shape[0]
    mesh = plsc.VectorSubcoreMesh(core_axis_name="c", subcore_axis_name="s")

    @functools.partial(
        pl.kernel, mesh=mesh,
        out_type=[
            jax.ShapeDtypeStruct((BN,), jnp.float32),
            jax.ShapeDtypeStruct((BN,), jnp.float32),
        ],
        scratch_types=[
            pltpu.VMEM((_QW,), jnp.float32),
            pltpu.VMEM((_QW,), jnp.float32),
            pltpu.VMEM((ksc,), jnp.float32),
            pltpu.VMEM((ksc,), jnp.float32),
            pltpu.VMEM((ksc,), jnp.float32),
            pltpu.VMEM((_QW,), jnp.float32),
            pltpu.VMEM((_QW,), jnp.float32),
        ],
    )
    def k(px_h, py_h, kx_h, ky_h, val_h, idx_h, qx_v, qy_v, kxr_v, kyr_v,
          b2_v, val_v, idx_v):
        wid = lax.axis_index("s") * _NC + lax.axis_index("c")
        qbase = wid * _QW
        batch = wid // 8
        pltpu.sync_copy(px_h.at[pl.ds(qbase, _QW)], qx_v)
        pltpu.sync_copy(py_h.at[pl.ds(qbase, _QW)], qy_v)
        kbase = batch * ksc
        pltpu.sync_copy(kx_h.at[pl.ds(kbase, ksc)], kxr_v)
        pltpu.sync_copy(ky_h.at[pl.ds(kbase, ksc)], kyr_v)

        def kprep(j, _):
            kxv = kxr_v[pl.ds(j * 16, 16)]
            kyv = kyr_v[pl.ds(j * 16, 16)]
            b2_v[pl.ds(j * 16, 16)] = kxv * kxv + kyv * kyv
            kxr_v[pl.ds(j * 16, 16)] = _round_bf16_vec(kxv)
            kyr_v[pl.ds(j * 16, 16)] = _round_bf16_vec(kyv)
            return 0
        lax.fori_loop(0, ksc // 16, kprep, 0)

        for qi in range(_QW // 16):
            qxv = qx_v[pl.ds(qi * 16, 16)]
            qyv = qy_v[pl.ds(qi * 16, 16)]
            a2 = qxv * qxv + qyv * qyv
            qmx = _round_bf16_vec(qxv) * -2.0
            qmy = _round_bf16_vec(qyv) * -2.0

            def body(jo, carry):
                rv, ri = carry
                base = jo * 16
                kxv = kxr_v[pl.ds(base, 16)]
                kyv = kyr_v[pl.ds(base, 16)]
                b2v = b2_v[pl.ds(base, 16)]
                jf_base = base.astype(jnp.float32)
                for u in range(16):
                    p = qmx * kxv[u] + qmy * kyv[u]
                    d2 = (a2 + b2v[u]) + p
                    c = jnp.maximum(d2, 0.0)
                    lt = c < rv
                    rv = jnp.where(lt, c, rv)
                    ri = jnp.where(lt, jf_base + float(u), ri)
                return rv, ri

            rv, ri = lax.fori_loop(
                0, ksc // 16, body,
                (jnp.full((16,), _BIGF, jnp.float32),
                 jnp.zeros((16,), jnp.float32)))
            val_v[pl.ds(qi * 16, 16)] = rv
            idx_v[pl.ds(qi * 16, 16)] = ri

        pltpu.sync_copy(val_v, val_h.at[pl.ds(qbase, _QW)])
        pltpu.sync_copy(idx_v, idx_h.at[pl.ds(qbase, _QW)])

    return k(px, py, kx, ky)


_CH = 128


def _merge_gather_sc(table, tc_idx, tc_val, sc_idx, sc_val, n_keys, ksplit):
    rows, G = table.shape
    per_w = rows // _NW
    nch = per_w // _CH
    mesh = plsc.VectorSubcoreMesh(core_axis_name="c", subcore_axis_name="s")

    @functools.partial(
        pl.kernel, mesh=mesh,
        out_type=jax.ShapeDtypeStruct((rows, G), jnp.float32),
        scratch_types=[
            pltpu.VMEM((per_w,), jnp.int32),
            pltpu.VMEM((per_w,), jnp.float32),
            pltpu.VMEM((per_w,), jnp.float32),
            pltpu.VMEM((per_w,), jnp.float32),
            pltpu.VMEM((per_w,), jnp.float32),
            pltpu.VMEM((_CH, G), jnp.float32),
            pltpu.VMEM((_CH, G), jnp.float32),
            pltpu.SemaphoreType.DMA,
            pltpu.SemaphoreType.DMA,
        ],
    )
    def k(table_hbm, tci_h, tcv_h, sci_h, scv_h, out_hbm, idx_v, tci_v,
          tcv_v, sci_v, scv_v, buf0, buf1, sem0, sem1):
        wid = lax.axis_index("s") * _NC + lax.axis_index("c")
        base = wid * per_w
        batch = (wid * per_w) // 4096
        pltpu.sync_copy(tci_h.at[pl.ds(base, per_w)], tci_v)
        pltpu.sync_copy(tcv_h.at[pl.ds(base, per_w)], tcv_v)
        pltpu.sync_copy(sci_h.at[pl.ds(base, per_w)], sci_v)
        pltpu.sync_copy(scv_h.at[pl.ds(base, per_w)], scv_v)
        off = (batch * n_keys + ksplit).astype(jnp.float32)
        for i in range(per_w // 16):
            tv = tcv_v[pl.ds(i * 16, 16)]
            sv = scv_v[pl.ds(i * 16, 16)]
            ti = tci_v[pl.ds(i * 16, 16)]
            si = sci_v[pl.ds(i * 16, 16)] + off
            use_tc = tv <= sv
            idx_v[pl.ds(i * 16, 16)] = jnp.where(use_tc, ti, si).astype(jnp.int32)
        bufs = (buf0, buf1)
        sems = (sem0, sem1)
        cps = []
        for c in range(nch):
            cps.append(pltpu.async_copy(
                table_hbm.at[idx_v.at[pl.ds(c * _CH, _CH)]],
                bufs[c % 2], sems[c % 2]))
            if c >= 1:
                cps[c - 1].wait()
                pltpu.sync_copy(bufs[(c - 1) % 2],
                                out_hbm.at[pl.ds(base + (c - 1) * _CH, _CH)])
        cps[nch - 1].wait()
        pltpu.sync_copy(bufs[(nch - 1) % 2],
                        out_hbm.at[pl.ds(base + (nch - 1) * _CH, _CH)])

    return k(table, tc_idx, tc_val, sc_idx, sc_val)


def kernel(predicted_positions, real_positions, real_expressions):
    B, N, _ = predicted_positions.shape
    G = real_expressions.shape[2]
    ksc = N - _KSPLIT
    qs = (predicted_positions * -2.0).astype(jnp.bfloat16)
    kt = real_positions.transpose(0, 2, 1)
    kt_tc = kt[:, :, :_KSPLIT]
    ks_tc = kt_tc.astype(jnp.bfloat16)
    px = predicted_positions[:, :, 0].reshape(B * N)
    py = predicted_positions[:, :, 1].reshape(B * N)
    kx = real_positions[:, _KSPLIT:, 0].reshape(B * ksc)
    ky = real_positions[:, _KSPLIT:, 1].reshape(B * ksc)
    sc_val, sc_idx = _nearest_sc(px, py, kx, ky, N, ksc)
    tc_idx, tc_val = _nearest_tc(predicted_positions, qs, ks_tc, kt_tc)
    tc_idx_f = tc_idx.astype(jnp.float32)
    table = real_expressions.reshape(B * N, G)
    out = _merge_gather_sc(table, tc_idx_f, tc_val, sc_idx, sc_val, N, _KSPLIT)
    return out.reshape(B, N, G)
